# baseline (speedup 1.0000x reference)
import functools

import jax
import jax.numpy as jnp
from jax import lax
from jax.experimental import pallas as pl
from jax.experimental.pallas import tpu as pltpu
from jax.experimental.pallas import tpu_sc as plsc

_LANES = 16
_CHUNK = 128


def _sc_gather_add(table1, table2, idx1f, idx2f):
    n_rows = idx1f.shape[0]
    embed_dim = table1.shape[1]
    info = plsc.get_sparse_core_info()
    nc, ns = info.num_cores, info.num_subcores
    nw = nc * ns
    per_w = n_rows // nw
    n_chunks = per_w // _CHUNK

    mesh = plsc.VectorSubcoreMesh(core_axis_name="c", subcore_axis_name="s")

    @functools.partial(
        pl.kernel,
        mesh=mesh,
        out_type=jax.ShapeDtypeStruct((n_rows, embed_dim), jnp.float32),
        scratch_types=[
            pltpu.VMEM((per_w,), jnp.int32),
            pltpu.VMEM((per_w,), jnp.int32),
            pltpu.VMEM((_CHUNK, embed_dim), jnp.float32),
            pltpu.VMEM((_CHUNK, embed_dim), jnp.float32),
            pltpu.SemaphoreType.DMA,
            pltpu.SemaphoreType.DMA,
        ],
        compiler_params=pltpu.CompilerParams(use_tc_tiling_on_sc=False),
    )
    def gather_kernel(t1, t2, i1, i2, out, i1_v, i2_v, r1, r2, s1, s2):
        wid = lax.axis_index("s") * nc + lax.axis_index("c")
        base = wid * per_w
        pltpu.sync_copy(i1.at[pl.ds(base, per_w)], i1_v)
        pltpu.sync_copy(i2.at[pl.ds(base, per_w)], i2_v)

        def chunk(g, carry):
            c1 = pltpu.async_copy(
                t1.at[i1_v.at[pl.ds(g * _CHUNK, _CHUNK)]], r1, s1)
            c2 = pltpu.async_copy(
                t2.at[i2_v.at[pl.ds(g * _CHUNK, _CHUNK)]], r2, s2)
            c1.wait()
            c2.wait()

            def add_row(i, c):
                for j in range(embed_dim // _LANES):
                    sl = (i, pl.ds(j * _LANES, _LANES))
                    r1[sl] = r1[sl] + r2[sl]
                return c

            lax.fori_loop(0, _CHUNK, add_row, 0)
            pltpu.sync_copy(r1, out.at[pl.ds(base + g * _CHUNK, _CHUNK)])
            return carry

        lax.fori_loop(0, n_chunks, chunk, 0)

    return gather_kernel(table1, table2, idx1f, idx2f)


def _tc_mlp(emb, W1, W2, b2, B, L):
    n_rows, D = emb.shape
    H = W1.shape[0]
    blk_b = 128
    blk_r = blk_b * L
    nb = B // blk_b
    w2col = W2.reshape(H, 1)
    b2m = b2.reshape(1, 1)
    gmat = (jnp.arange(blk_r, dtype=jnp.int32)[:, None] // L
            == jnp.arange(blk_b, dtype=jnp.int32)[None, :]
            ).astype(jnp.float32)

    def body(e_ref, w1_ref, w2_ref, b2_ref, g_ref, o_ref):
        x = e_ref[...]
        ht = lax.dot_general(
            w1_ref[...], x, (((1,), (1,)), ((), ())),
            preferred_element_type=jnp.float32)
        ht = jnp.maximum(ht, 0.0)
        y = jnp.sum(ht * w2_ref[...], axis=0, keepdims=True)
        o = lax.dot_general(
            y, g_ref[...], (((1,), (0,)), ((), ())),
            preferred_element_type=jnp.float32)
        o_ref[...] = o + L * b2_ref[0, 0]

    out_row = pl.pallas_call(
        body,
        grid=(nb,),
        in_specs=[
            pl.BlockSpec((blk_r, D), lambda i: (i, 0)),
            pl.BlockSpec((H, D), lambda i: (0, 0)),
            pl.BlockSpec((H, 1), lambda i: (0, 0)),
            pl.BlockSpec((1, 1), lambda i: (0, 0)),
            pl.BlockSpec((blk_r, blk_b), lambda i: (0, 0)),
        ],
        out_specs=pl.BlockSpec((1, blk_b), lambda i: (0, i)),
        out_shape=jax.ShapeDtypeStruct((1, B), jnp.float32),
    )(emb, W1, w2col, b2m, gmat)
    return out_row.reshape(B, 1)


def _linearize(t):
    v, d = t.shape
    lines = lax.optimization_barrier(t.reshape(v * d // 128, 128))
    return lines.reshape(v, d)


def kernel(input1, input2, table1, table2, W1, W2, b2):
    B, L = input1.shape
    n_rows = B * L

    idx1f = input1.astype(jnp.int32).reshape(n_rows)
    idx2f = input2.astype(jnp.int32).reshape(n_rows)

    emb = _sc_gather_add(_linearize(table1), _linearize(table2), idx1f, idx2f)
    return _tc_mlp(emb, W1, W2, b2, B, L)


# speedup vs baseline: 1.4549x; 1.4549x over previous
import functools

import jax
import jax.numpy as jnp
from jax import lax
from jax.experimental import pallas as pl
from jax.experimental.pallas import tpu as pltpu
from jax.experimental.pallas import tpu_sc as plsc

_LANES = 16
_CHUNK = 128


def _sc_gather_add(table1, table2, idx1f, idx2f):
    n_rows = idx1f.shape[0]
    embed_dim = table1.shape[1]
    info = plsc.get_sparse_core_info()
    nc, ns = info.num_cores, info.num_subcores
    nw = nc * ns
    per_w = n_rows // nw
    n_chunks = per_w // _CHUNK

    mesh = plsc.VectorSubcoreMesh(core_axis_name="c", subcore_axis_name="s")

    @functools.partial(
        pl.kernel,
        mesh=mesh,
        out_type=jax.ShapeDtypeStruct((n_rows, embed_dim), jnp.float32),
        scratch_types=[
            pltpu.VMEM((per_w,), jnp.int32),
            pltpu.VMEM((per_w,), jnp.int32),
            pltpu.VMEM((_CHUNK, embed_dim), jnp.float32),
            pltpu.VMEM((_CHUNK, embed_dim), jnp.float32),
            pltpu.SemaphoreType.DMA,
            pltpu.SemaphoreType.DMA,
        ],
        compiler_params=pltpu.CompilerParams(use_tc_tiling_on_sc=False),
    )
    def gather_kernel(t1, t2, i1, i2, out, i1_v, i2_v, r1, r2, s1, s2):
        wid = lax.axis_index("s") * nc + lax.axis_index("c")
        base = wid * per_w
        pltpu.sync_copy(i1.at[pl.ds(base, per_w)], i1_v)
        pltpu.sync_copy(i2.at[pl.ds(base, per_w)], i2_v)

        def chunk(g, carry):
            c1 = pltpu.async_copy(
                t1.at[i1_v.at[pl.ds(g * _CHUNK, _CHUNK)]], r1, s1)
            c2 = pltpu.async_copy(
                t2.at[i2_v.at[pl.ds(g * _CHUNK, _CHUNK)]], r2, s2)
            c1.wait()
            c2.wait()

            def add_row(i, c):
                for j in range(embed_dim // _LANES):
                    sl = (i, pl.ds(j * _LANES, _LANES))
                    r1[sl] = r1[sl] + r2[sl]
                return c

            lax.fori_loop(0, _CHUNK, add_row, 0)
            pltpu.sync_copy(r1, out.at[pl.ds(base + g * _CHUNK, _CHUNK)])
            return carry

        lax.fori_loop(0, n_chunks, chunk, 0)

    return gather_kernel(table1, table2, idx1f, idx2f)


def _tc_mlp(emb, W1, W2, b2, B, L):
    n_rows, D = emb.shape
    H = W1.shape[0]
    blk_b = 128
    blk_r = blk_b * L
    nb = B // blk_b
    w2col = W2.reshape(H, 1)
    b2m = b2.reshape(1, 1)
    gmat = (jnp.arange(blk_r, dtype=jnp.int32)[:, None] // L
            == jnp.arange(blk_b, dtype=jnp.int32)[None, :]
            ).astype(jnp.float32)

    def body(e_ref, w1_ref, w2_ref, b2_ref, g_ref, o_ref):
        x = e_ref[...]
        ht = lax.dot_general(
            w1_ref[...], x, (((1,), (1,)), ((), ())),
            preferred_element_type=jnp.float32)
        ht = jnp.maximum(ht, 0.0)
        y = jnp.sum(ht * w2_ref[...], axis=0, keepdims=True)
        o = lax.dot_general(
            y, g_ref[...], (((1,), (0,)), ((), ())),
            preferred_element_type=jnp.float32)
        o_ref[...] = o + L * b2_ref[0, 0]

    out_row = pl.pallas_call(
        body,
        grid=(nb,),
        in_specs=[
            pl.BlockSpec((blk_r, D), lambda i: (i, 0)),
            pl.BlockSpec((H, D), lambda i: (0, 0)),
            pl.BlockSpec((H, 1), lambda i: (0, 0)),
            pl.BlockSpec((1, 1), lambda i: (0, 0)),
            pl.BlockSpec((blk_r, blk_b), lambda i: (0, 0)),
        ],
        out_specs=pl.BlockSpec((1, blk_b), lambda i: (0, i)),
        out_shape=jax.ShapeDtypeStruct((1, B), jnp.float32),
    )(emb, W1, w2col, b2m, gmat)
    return out_row.reshape(B, 1)


def _linearize(t):
    tt = t.T
    D, V = tt.shape
    J = 2048
    n_full = V // (4 * J)
    nsteps = n_full + 1
    m_full = 4 * n_full
    tail = jnp.pad(tt[:, m_full * J:], ((0, 0), (0, J - V % J)))

    def body(x0, x1, x2, x3, tl, o_ref):
        i = pl.program_id(0)
        eye = (lax.broadcasted_iota(jnp.int32, (D, D), 0)
               == lax.broadcasted_iota(jnp.int32, (D, D), 1)
               ).astype(jnp.float32)

        def tp(x):
            return lax.dot_general(x, eye, (((0,), (0,)), ((), ())),
                                   preferred_element_type=jnp.float32)

        @pl.when(i < n_full)
        def _main():
            for o, xr in enumerate((x0, x1, x2, x3)):
                o_ref[:, o * D:(o + 1) * D] = tp(xr[...])

        @pl.when(i == n_full)
        def _tail():
            o_ref[:, :D] = tp(tl[...])
            o_ref[:, D:] = jnp.zeros((J, 3 * D), jnp.float32)

    lines = pl.pallas_call(
        body,
        grid=(nsteps,),
        in_specs=[
            pl.BlockSpec(
                (D, J), lambda i, o=o: (0, jnp.minimum(4 * i + o, m_full - 1)))
            for o in range(4)
        ] + [pl.BlockSpec((D, J), lambda i: (0, 0))],
        out_specs=pl.BlockSpec((J, 4 * D), lambda i: (i, 0)),
        out_shape=jax.ShapeDtypeStruct((nsteps * J, 4 * D), jnp.float32),
    )(tt, tt, tt, tt, tail)
    return lines.reshape(nsteps * J * 4, D), J


def kernel(input1, input2, table1, table2, W1, W2, b2):
    B, L = input1.shape
    n_rows = B * L

    t1lin, j1 = _linearize(table1)
    t2lin, j2 = _linearize(table2)

    def remap(v, j):
        m, r = v // j, v % j
        return (m // 4) * (4 * j) + r * 4 + m % 4

    idx1f = remap(input1.astype(jnp.int32).reshape(n_rows), j1)
    idx2f = remap(input2.astype(jnp.int32).reshape(n_rows), j2)

    emb = _sc_gather_add(t1lin, t2lin, idx1f, idx2f)
    return _tc_mlp(emb, W1, W2, b2, B, L)


# speedup vs baseline: 2.1241x; 1.4600x over previous
import functools

import jax
import jax.numpy as jnp
from jax import lax
from jax.experimental import pallas as pl
from jax.experimental.pallas import tpu as pltpu
from jax.experimental.pallas import tpu_sc as plsc

_LANES = 16
_CHUNK = 128


def _sc_gather_add(table1, table2, idx1f, idx2f):
    n_rows = idx1f.shape[0]
    embed_dim = table1.shape[1]
    info = plsc.get_sparse_core_info()
    nc, ns = info.num_cores, info.num_subcores
    nw = nc * ns
    per_w = n_rows // nw
    n_chunks = per_w // _CHUNK

    mesh = plsc.VectorSubcoreMesh(core_axis_name="c", subcore_axis_name="s")

    @functools.partial(
        pl.kernel,
        mesh=mesh,
        out_type=jax.ShapeDtypeStruct((n_rows, embed_dim), jnp.float32),
        scratch_types=[
            pltpu.VMEM((per_w,), jnp.int32),
            pltpu.VMEM((per_w,), jnp.int32),
            pltpu.VMEM((_CHUNK, embed_dim), jnp.float32),
            pltpu.VMEM((_CHUNK, embed_dim), jnp.float32),
            pltpu.SemaphoreType.DMA,
            pltpu.SemaphoreType.DMA,
        ],
        compiler_params=pltpu.CompilerParams(use_tc_tiling_on_sc=False),
    )
    def gather_kernel(t1, t2, i1, i2, out, i1_v, i2_v, r1, r2, s1, s2):
        wid = lax.axis_index("s") * nc + lax.axis_index("c")
        base = wid * per_w
        pltpu.sync_copy(i1.at[pl.ds(base, per_w)], i1_v)
        pltpu.sync_copy(i2.at[pl.ds(base, per_w)], i2_v)

        def chunk(g, carry):
            c1 = pltpu.async_copy(
                t1.at[i1_v.at[pl.ds(g * _CHUNK, _CHUNK)]], r1, s1)
            c2 = pltpu.async_copy(
                t2.at[i2_v.at[pl.ds(g * _CHUNK, _CHUNK)]], r2, s2)
            c1.wait()
            c2.wait()

            def add_row(i, c):
                for j in range(embed_dim // _LANES):
                    sl = (i, pl.ds(j * _LANES, _LANES))
                    r1[sl] = r1[sl] + r2[sl]
                return c

            lax.fori_loop(0, _CHUNK, add_row, 0)
            pltpu.sync_copy(r1, out.at[pl.ds(base + g * _CHUNK, _CHUNK)])
            return carry

        lax.fori_loop(0, n_chunks, chunk, 0)

    return gather_kernel(table1, table2, idx1f, idx2f)


def _tc_mlp(emb, W1, W2, b2, B, L):
    n_rows, D = emb.shape
    H = W1.shape[0]
    blk_b = 128
    blk_r = blk_b * L
    nb = B // blk_b
    w2col = W2.reshape(H, 1)
    b2m = b2.reshape(1, 1)
    gmat = (jnp.arange(blk_r, dtype=jnp.int32)[:, None] // L
            == jnp.arange(blk_b, dtype=jnp.int32)[None, :]
            ).astype(jnp.float32)

    def body(e_ref, w1_ref, w2_ref, b2_ref, g_ref, o_ref):
        x = e_ref[...]
        ht = lax.dot_general(
            w1_ref[...], x, (((1,), (1,)), ((), ())),
            preferred_element_type=jnp.float32)
        ht = jnp.maximum(ht, 0.0)
        y = jnp.sum(ht * w2_ref[...], axis=0, keepdims=True)
        o = lax.dot_general(
            y, g_ref[...], (((1,), (0,)), ((), ())),
            preferred_element_type=jnp.float32)
        o_ref[...] = o + L * b2_ref[0, 0]

    out_row = pl.pallas_call(
        body,
        grid=(nb,),
        in_specs=[
            pl.BlockSpec((blk_r, D), lambda i: (i, 0)),
            pl.BlockSpec((H, D), lambda i: (0, 0)),
            pl.BlockSpec((H, 1), lambda i: (0, 0)),
            pl.BlockSpec((1, 1), lambda i: (0, 0)),
            pl.BlockSpec((blk_r, blk_b), lambda i: (0, 0)),
        ],
        out_specs=pl.BlockSpec((1, blk_b), lambda i: (0, i)),
        out_shape=jax.ShapeDtypeStruct((1, B), jnp.float32),
    )(emb, W1, w2col, b2m, gmat)
    return out_row.reshape(B, 1)


def _linearize(t):
    tt = t.T
    D, V = tt.shape
    J = 2048
    n_full = V // (4 * J)
    nsteps = n_full + 1
    m_full = 4 * n_full
    tail = jnp.pad(tt[:, m_full * J:], ((0, 0), (0, J - V % J)))

    def body(x0, x1, x2, x3, tl, o_ref):
        i = pl.program_id(0)

        @pl.when(i < n_full)
        def _main():
            xcat = jnp.concatenate(
                [x0[...], x1[...], x2[...], x3[...]], axis=0)
            o_ref[...] = jnp.swapaxes(xcat, 0, 1)

        @pl.when(i == n_full)
        def _tail():
            xcat = jnp.concatenate(
                [tl[...], jnp.zeros((3 * D, J), jnp.float32)], axis=0)
            o_ref[...] = jnp.swapaxes(xcat, 0, 1)

    lines = pl.pallas_call(
        body,
        grid=(nsteps,),
        in_specs=[
            pl.BlockSpec(
                (D, J), lambda i, o=o: (0, jnp.minimum(4 * i + o, m_full - 1)))
            for o in range(4)
        ] + [pl.BlockSpec((D, J), lambda i: (0, 0))],
        out_specs=pl.BlockSpec((J, 4 * D), lambda i: (i, 0)),
        out_shape=jax.ShapeDtypeStruct((nsteps * J, 4 * D), jnp.float32),
    )(tt, tt, tt, tt, tail)
    return lines.reshape(nsteps * J * 4, D), J


def kernel(input1, input2, table1, table2, W1, W2, b2):
    B, L = input1.shape
    n_rows = B * L

    t1lin, j1 = _linearize(table1)
    t2lin, j2 = _linearize(table2)

    def remap(v, j):
        m, r = v // j, v % j
        return (m // 4) * (4 * j) + r * 4 + m % 4

    idx1f = remap(input1.astype(jnp.int32).reshape(n_rows), j1)
    idx2f = remap(input2.astype(jnp.int32).reshape(n_rows), j2)

    emb = _sc_gather_add(t1lin, t2lin, idx1f, idx2f)
    return _tc_mlp(emb, W1, W2, b2, B, L)


# speedup vs baseline: 2.7224x; 1.2817x over previous
import functools

import jax
import jax.numpy as jnp
from jax import lax
from jax.experimental import pallas as pl
from jax.experimental.pallas import tpu as pltpu
from jax.experimental.pallas import tpu_sc as plsc

_LANES = 16
_CHUNK = 128


def _sc_gather(table, idxf):
    n_rows = idxf.shape[0]
    embed_dim = table.shape[1]
    info = plsc.get_sparse_core_info()
    nc, ns = info.num_cores, info.num_subcores
    nw = nc * ns
    per_w = n_rows // nw
    n_chunks = per_w // _CHUNK

    mesh = plsc.VectorSubcoreMesh(core_axis_name="c", subcore_axis_name="s")

    @functools.partial(
        pl.kernel,
        mesh=mesh,
        out_type=jax.ShapeDtypeStruct((n_rows, embed_dim), jnp.float32),
        scratch_types=[
            pltpu.VMEM((per_w,), jnp.int32),
            pltpu.VMEM((_CHUNK, embed_dim), jnp.float32),
            pltpu.VMEM((_CHUNK, embed_dim), jnp.float32),
            pltpu.SemaphoreType.DMA,
            pltpu.SemaphoreType.DMA,
        ],
        compiler_params=pltpu.CompilerParams(use_tc_tiling_on_sc=False),
    )
    def gather_kernel(t, iref, out, i_v, r0, r1, s0, s1):
        wid = lax.axis_index("s") * nc + lax.axis_index("c")
        base = wid * per_w
        pltpu.sync_copy(iref.at[pl.ds(base, per_w)], i_v)

        bufs = (r0, r1)
        sems = (s0, s1)

        def gather_chunk(g, buf, sem):
            return pltpu.async_copy(
                t.at[i_v.at[pl.ds(g * _CHUNK, _CHUNK)]], buf, sem)

        copies = [gather_chunk(0, r0, s0)]
        for g in range(n_chunks):
            if g + 1 < n_chunks:
                copies.append(
                    gather_chunk(g + 1, bufs[(g + 1) % 2], sems[(g + 1) % 2]))
            copies[g].wait()
            pltpu.sync_copy(
                bufs[g % 2], out.at[pl.ds(base + g * _CHUNK, _CHUNK)])

    return gather_kernel(table, idxf)


def _tc_mlp(emb1, emb2, W1, W2, b2, B, L):
    H, D = W1.shape
    blk_b = 128
    blk_r = blk_b * L
    blk_j = blk_r // 4
    nb = B // blk_b
    w2col = W2.reshape(H, 1)
    b2m = b2.reshape(1, 1)
    jj = jnp.arange(blk_j, dtype=jnp.int32)[:, None, None]
    oo = jnp.arange(4, dtype=jnp.int32)[None, :, None]
    ii = jnp.arange(blk_b, dtype=jnp.int32)[None, None, :]
    gmat = ((4 * jj + oo) // L == ii).astype(jnp.float32).reshape(
        blk_j, 4 * blk_b)

    def body(e1_ref, e2_ref, w1_ref, w2_ref, b2_ref, g_ref, o_ref):
        x = e1_ref[...] + e2_ref[...]
        o = None
        for p in range(4):
            xo = x[:, p * D:(p + 1) * D]
            ht = lax.dot_general(
                w1_ref[...], xo, (((1,), (1,)), ((), ())),
                preferred_element_type=jnp.float32)
            ht = jnp.maximum(ht, 0.0)
            y = jnp.sum(ht * w2_ref[...], axis=0, keepdims=True)
            go = g_ref[:, p * blk_b:(p + 1) * blk_b]
            contrib = lax.dot_general(
                y, go, (((1,), (0,)), ((), ())),
                preferred_element_type=jnp.float32)
            o = contrib if o is None else o + contrib
        o_ref[...] = o + L * b2_ref[0, 0]

    out_row = pl.pallas_call(
        body,
        grid=(nb,),
        in_specs=[
            pl.BlockSpec((blk_j, 128), lambda i: (i, 0)),
            pl.BlockSpec((blk_j, 128), lambda i: (i, 0)),
            pl.BlockSpec((H, D), lambda i: (0, 0)),
            pl.BlockSpec((H, 1), lambda i: (0, 0)),
            pl.BlockSpec((1, 1), lambda i: (0, 0)),
            pl.BlockSpec((blk_j, 4 * blk_b), lambda i: (0, 0)),
        ],
        out_specs=pl.BlockSpec((1, blk_b), lambda i: (0, i)),
        out_shape=jax.ShapeDtypeStruct((1, B), jnp.float32),
    )(emb1, emb2, W1, w2col, b2m, gmat)
    return out_row.reshape(B, 1)


def _linearize(t):
    tt = t.T
    D, V = tt.shape
    J = 2048
    n_full = V // (4 * J)
    nsteps = n_full + 1
    m_full = 4 * n_full
    tail = jnp.pad(tt[:, m_full * J:], ((0, 0), (0, J - V % J)))

    def body(x0, x1, x2, x3, tl, o_ref):
        i = pl.program_id(0)

        @pl.when(i < n_full)
        def _main():
            xcat = jnp.concatenate(
                [x0[...], x1[...], x2[...], x3[...]], axis=0)
            o_ref[...] = jnp.swapaxes(xcat, 0, 1)

        @pl.when(i == n_full)
        def _tail():
            xcat = jnp.concatenate(
                [tl[...], jnp.zeros((3 * D, J), jnp.float32)], axis=0)
            o_ref[...] = jnp.swapaxes(xcat, 0, 1)

    lines = pl.pallas_call(
        body,
        grid=(nsteps,),
        in_specs=[
            pl.BlockSpec(
                (D, J), lambda i, o=o: (0, jnp.minimum(4 * i + o, m_full - 1)))
            for o in range(4)
        ] + [pl.BlockSpec((D, J), lambda i: (0, 0))],
        out_specs=pl.BlockSpec((J, 4 * D), lambda i: (i, 0)),
        out_shape=jax.ShapeDtypeStruct((nsteps * J, 4 * D), jnp.float32),
    )(tt, tt, tt, tt, tail)
    return lines.reshape(nsteps * J * 4, D), J


def kernel(input1, input2, table1, table2, W1, W2, b2):
    B, L = input1.shape
    n_rows = B * L

    t1lin, j1 = _linearize(table1)
    t2lin, j2 = _linearize(table2)

    def remap(v, j):
        m, r = v // j, v % j
        return (m // 4) * (4 * j) + r * 4 + m % 4

    idx1f = remap(input1.astype(jnp.int32).reshape(n_rows), j1)
    idx2f = remap(input2.astype(jnp.int32).reshape(n_rows), j2)

    emb1 = _sc_gather(t1lin, idx1f).reshape(n_rows // 4, 128)
    emb2 = _sc_gather(t2lin, idx2f).reshape(n_rows // 4, 128)
    return _tc_mlp(emb1, emb2, W1, W2, b2, B, L)


# speedup vs baseline: 3.3018x; 1.2128x over previous
import functools

import jax
import jax.numpy as jnp
from jax import lax
from jax.experimental import pallas as pl
from jax.experimental.pallas import tpu as pltpu
from jax.experimental.pallas import tpu_sc as plsc

_LANES = 16
_CHUNK = 128


def _sc_gather(table, idxf):
    n_rows = idxf.shape[0]
    embed_dim = table.shape[1]
    info = plsc.get_sparse_core_info()
    nc, ns = info.num_cores, info.num_subcores
    nw = nc * ns
    per_w = n_rows // nw
    n_chunks = per_w // _CHUNK

    mesh = plsc.VectorSubcoreMesh(core_axis_name="c", subcore_axis_name="s")

    @functools.partial(
        pl.kernel,
        mesh=mesh,
        out_type=jax.ShapeDtypeStruct((n_rows, embed_dim), jnp.float32),
        scratch_types=[
            pltpu.VMEM((per_w,), jnp.int32),
            pltpu.VMEM((_CHUNK, embed_dim), jnp.float32),
            pltpu.VMEM((_CHUNK, embed_dim), jnp.float32),
            pltpu.SemaphoreType.DMA,
            pltpu.SemaphoreType.DMA,
        ],
        compiler_params=pltpu.CompilerParams(use_tc_tiling_on_sc=False),
    )
    def gather_kernel(t, iref, out, i_v, r0, r1, s0, s1):
        wid = lax.axis_index("s") * nc + lax.axis_index("c")
        base = wid * per_w
        pltpu.sync_copy(iref.at[pl.ds(base, per_w)], i_v)

        bufs = (r0, r1)
        sems = (s0, s1)

        def gather_chunk(g, buf, sem):
            return pltpu.async_copy(
                t.at[i_v.at[pl.ds(g * _CHUNK, _CHUNK)]], buf, sem)

        copies = [gather_chunk(0, r0, s0)]
        for g in range(n_chunks):
            if g + 1 < n_chunks:
                copies.append(
                    gather_chunk(g + 1, bufs[(g + 1) % 2], sems[(g + 1) % 2]))
            copies[g].wait()
            pltpu.sync_copy(
                bufs[g % 2], out.at[pl.ds(base + g * _CHUNK, _CHUNK)])

    return gather_kernel(table, idxf)


def _tc_mlp(emb1, emb2, W1, W2, b2, B, L):
    H, D = W1.shape
    blk_b = 128
    blk_r = blk_b * L
    blk_j = blk_r // 4
    nb = B // blk_b
    w2col = W2.reshape(H, 1)
    b2m = b2.reshape(1, 1)
    jj = jnp.arange(blk_j, dtype=jnp.int32)[:, None, None]
    oo = jnp.arange(4, dtype=jnp.int32)[None, :, None]
    ii = jnp.arange(blk_b, dtype=jnp.int32)[None, None, :]
    gmat = ((4 * jj + oo) // L == ii).astype(jnp.float32).reshape(
        blk_j, 4 * blk_b)

    def body(e1_ref, e2_ref, w1_ref, w2_ref, b2_ref, g_ref, o_ref):
        x = e1_ref[...] + e2_ref[...]
        o = None
        for p in range(4):
            xo = x[:, p * D:(p + 1) * D]
            ht = lax.dot_general(
                w1_ref[...], xo, (((1,), (1,)), ((), ())),
                preferred_element_type=jnp.float32)
            ht = jnp.maximum(ht, 0.0)
            y = jnp.sum(ht * w2_ref[...], axis=0, keepdims=True)
            go = g_ref[:, p * blk_b:(p + 1) * blk_b]
            contrib = lax.dot_general(
                y, go, (((1,), (0,)), ((), ())),
                preferred_element_type=jnp.float32)
            o = contrib if o is None else o + contrib
        o_ref[...] = o + L * b2_ref[0, 0]

    out_row = pl.pallas_call(
        body,
        grid=(nb,),
        in_specs=[
            pl.BlockSpec((blk_j, 128), lambda i: (i, 0)),
            pl.BlockSpec((blk_j, 128), lambda i: (i, 0)),
            pl.BlockSpec((H, D), lambda i: (0, 0)),
            pl.BlockSpec((H, 1), lambda i: (0, 0)),
            pl.BlockSpec((1, 1), lambda i: (0, 0)),
            pl.BlockSpec((blk_j, 4 * blk_b), lambda i: (0, 0)),
        ],
        out_specs=pl.BlockSpec((1, blk_b), lambda i: (0, i)),
        out_shape=jax.ShapeDtypeStruct((1, B), jnp.float32),
    )(emb1, emb2, W1, w2col, b2m, gmat)
    return out_row.reshape(B, 1)


def _linearize(t):
    tt = t.T
    D, V = tt.shape
    J = 4096
    n_full = V // (4 * J)
    nsteps = n_full + 1
    m_full = 4 * n_full
    tail = jnp.pad(tt[:, m_full * J:], ((0, 0), (0, J - V % J)))

    def body(x0, x1, x2, x3, tl, o_ref):
        i = pl.program_id(0)

        @pl.when(i < n_full)
        def _main():
            xcat = jnp.concatenate(
                [x0[...], x1[...], x2[...], x3[...]], axis=0)
            o_ref[...] = jnp.swapaxes(xcat, 0, 1)

        @pl.when(i == n_full)
        def _tail():
            xcat = jnp.concatenate(
                [tl[...], jnp.zeros((3 * D, J), jnp.float32)], axis=0)
            o_ref[...] = jnp.swapaxes(xcat, 0, 1)

    lines = pl.pallas_call(
        body,
        grid=(nsteps,),
        in_specs=[
            pl.BlockSpec(
                (D, J), lambda i, o=o: (0, jnp.minimum(4 * i + o, m_full - 1)))
            for o in range(4)
        ] + [pl.BlockSpec((D, J), lambda i: (0, 0))],
        out_specs=pl.BlockSpec((J, 4 * D), lambda i: (i, 0)),
        out_shape=jax.ShapeDtypeStruct((nsteps * J, 4 * D), jnp.float32),
    )(tt, tt, tt, tt, tail)
    return lines.reshape(nsteps * J * 4, D), J


def kernel(input1, input2, table1, table2, W1, W2, b2):
    B, L = input1.shape
    n_rows = B * L

    t1lin, j1 = _linearize(table1)
    t2lin, j2 = _linearize(table2)

    def remap(v, j):
        m, r = v // j, v % j
        return (m // 4) * (4 * j) + r * 4 + m % 4

    idx1f = remap(input1.astype(jnp.int32).reshape(n_rows), j1)
    idx2f = remap(input2.astype(jnp.int32).reshape(n_rows), j2)

    emb1 = _sc_gather(t1lin, idx1f).reshape(n_rows // 4, 128)
    emb2 = _sc_gather(t2lin, idx2f).reshape(n_rows // 4, 128)
    return _tc_mlp(emb1, emb2, W1, W2, b2, B, L)
